# Initial kernel scaffold; baseline (speedup 1.0000x reference)
import jax
import jax.numpy as jnp
from jax.experimental import pallas as pl


def kernel(table, type_index):
    raise NotImplementedError("write your pallas kernel here")


# speedup vs baseline: 4.5301x; 4.5301x over previous
import functools

import jax
import jax.numpy as jnp
from jax import lax
from jax.experimental import pallas as pl
from jax.experimental.pallas import tpu as pltpu
from jax.experimental.pallas import tpu_sc as plsc

EMBED = 64
CHUNK = 128
NBUF = 2
NC = 2
NS = 16
NW = NC * NS


@functools.lru_cache(maxsize=None)
def _make_gather(n_rows: int):
    rows_per_w = n_rows // NW
    n_chunks = rows_per_w // CHUNK
    n_groups = n_chunks // NBUF
    assert rows_per_w * NW == n_rows and n_chunks * CHUNK == rows_per_w
    assert n_groups * NBUF == n_chunks
    mesh = plsc.VectorSubcoreMesh(core_axis_name="c", subcore_axis_name="s")

    @functools.partial(
        pl.kernel,
        mesh=mesh,
        compiler_params=pltpu.CompilerParams(use_tc_tiling_on_sc=False),
        out_type=jax.ShapeDtypeStruct((n_rows, EMBED), jnp.float32),
        scratch_types=(
            [pltpu.VMEM((n_chunks, CHUNK), jnp.int32)]
            + [pltpu.VMEM((CHUNK, EMBED), jnp.float32) for _ in range(NBUF)]
            + [pltpu.SemaphoreType.DMA for _ in range(NBUF)]
        ),
    )
    def gather(table_hbm, idx_hbm, out_hbm, idx_v, *bufs_and_sems):
        rows = bufs_and_sems[:NBUF]
        sems = bufs_and_sems[NBUF:]
        wid = lax.axis_index("s") * NC + lax.axis_index("c")
        base = wid * rows_per_w
        pltpu.sync_copy(idx_hbm.at[wid], idx_v)

        def start(b, c):
            pltpu.make_async_copy(
                table_hbm.at[idx_v.at[c]], rows[b], sems[b]
            ).start()

        for b in range(NBUF):
            start(b, b)

        def body(g, carry):
            for b in range(NBUF):
                c = g * NBUF + b
                pltpu.make_async_copy(
                    table_hbm.at[idx_v.at[c]], rows[b], sems[b]
                ).wait()
                pltpu.sync_copy(
                    rows[b], out_hbm.at[pl.ds(base + c * CHUNK, CHUNK)]
                )
                nxt = c + NBUF

                @pl.when(nxt < n_chunks)
                def _():
                    start(b, nxt)

            return carry

        lax.fori_loop(0, n_groups, body, 0)

    return gather


def kernel(table, type_index):
    bsz, hist = type_index.shape
    n_rows = bsz * hist
    idx = type_index.astype(jnp.int32).reshape(NW, n_rows // NW // CHUNK, CHUNK)
    out = _make_gather(n_rows)(table, idx)
    return out.reshape(bsz, hist, EMBED)


# speedup vs baseline: 4.6721x; 1.0313x over previous
import functools

import jax
import jax.numpy as jnp
from jax import lax
from jax.experimental import pallas as pl
from jax.experimental.pallas import tpu as pltpu
from jax.experimental.pallas import tpu_sc as plsc

EMBED = 64
CHUNK = 128
RING = 5
DEPTH = 3
NC = 2
NS = 16
NW = NC * NS


@functools.lru_cache(maxsize=None)
def _make_gather(n_rows: int):
    rows_per_w = n_rows // NW
    n_chunks = rows_per_w // CHUNK
    n_groups = n_chunks // RING
    assert rows_per_w * NW == n_rows and n_chunks * CHUNK == rows_per_w
    assert n_groups * RING == n_chunks
    mesh = plsc.VectorSubcoreMesh(core_axis_name="c", subcore_axis_name="s")

    @functools.partial(
        pl.kernel,
        mesh=mesh,
        compiler_params=pltpu.CompilerParams(use_tc_tiling_on_sc=False),
        out_type=jax.ShapeDtypeStruct((n_rows, EMBED), jnp.float32),
        scratch_types=(
            [pltpu.VMEM((n_chunks, CHUNK), jnp.int32)]
            + [pltpu.VMEM((CHUNK, EMBED), jnp.float32) for _ in range(RING)]
            + [pltpu.SemaphoreType.DMA for _ in range(2 * RING)]
        ),
    )
    def gather(table_hbm, idx_hbm, out_hbm, idx_v, *bufs_and_sems):
        rows = bufs_and_sems[:RING]
        gsem = bufs_and_sems[RING : 2 * RING]
        wsem = bufs_and_sems[2 * RING :]
        wid = lax.axis_index("s") * NC + lax.axis_index("c")
        base = wid * rows_per_w
        pltpu.sync_copy(idx_hbm.at[wid], idx_v)

        def gather_copy(b, c):
            return pltpu.make_async_copy(
                table_hbm.at[idx_v.at[c]], rows[b], gsem[b]
            )

        def write_copy(b, c):
            return pltpu.make_async_copy(
                rows[b], out_hbm.at[pl.ds(base + c * CHUNK, CHUNK)], wsem[b]
            )

        for b in range(DEPTH):
            gather_copy(b, b).start()

        def body(g, carry):
            for b in range(RING):
                c = g * RING + b
                gather_copy(b, c).wait()
                write_copy(b, c).start()
                bg = (b + DEPTH) % RING
                nxt = c + DEPTH

                @pl.when(nxt < n_chunks)
                def _():
                    @pl.when(nxt >= RING)
                    def _():
                        write_copy(bg, nxt - RING).wait()

                    gather_copy(bg, nxt).start()

            return carry

        lax.fori_loop(0, n_groups, body, 0)
        for b in range(RING):
            write_copy(b, n_chunks - RING + b).wait()

    return gather


def kernel(table, type_index):
    bsz, hist = type_index.shape
    n_rows = bsz * hist
    idx = type_index.astype(jnp.int32).reshape(NW, n_rows // NW // CHUNK, CHUNK)
    out = _make_gather(n_rows)(table, idx)
    return out.reshape(bsz, hist, EMBED)


# speedup vs baseline: 4.6914x; 1.0041x over previous
import functools

import jax
import jax.numpy as jnp
from jax import lax
from jax.experimental import pallas as pl
from jax.experimental.pallas import tpu as pltpu
from jax.experimental.pallas import tpu_sc as plsc

EMBED = 64
CHUNK = 128
RING = 10
DEPTH = 8
NC = 2
NS = 16
NW = NC * NS


@functools.lru_cache(maxsize=None)
def _make_gather(n_rows: int):
    rows_per_w = n_rows // NW
    n_chunks = rows_per_w // CHUNK
    n_groups = n_chunks // RING
    assert rows_per_w * NW == n_rows and n_chunks * CHUNK == rows_per_w
    assert n_groups * RING == n_chunks
    mesh = plsc.VectorSubcoreMesh(core_axis_name="c", subcore_axis_name="s")

    @functools.partial(
        pl.kernel,
        mesh=mesh,
        compiler_params=pltpu.CompilerParams(use_tc_tiling_on_sc=False),
        out_type=jax.ShapeDtypeStruct((n_rows, EMBED), jnp.float32),
        scratch_types=(
            [pltpu.VMEM((n_chunks, CHUNK), jnp.int32)]
            + [pltpu.VMEM((CHUNK, EMBED), jnp.float32) for _ in range(RING)]
            + [pltpu.SemaphoreType.DMA for _ in range(2 * RING)]
        ),
    )
    def gather(table_hbm, idx_hbm, out_hbm, idx_v, *bufs_and_sems):
        rows = bufs_and_sems[:RING]
        gsem = bufs_and_sems[RING : 2 * RING]
        wsem = bufs_and_sems[2 * RING :]
        wid = lax.axis_index("s") * NC + lax.axis_index("c")
        base = wid * rows_per_w
        pltpu.sync_copy(idx_hbm.at[wid], idx_v)

        def gather_copy(b, c):
            return pltpu.make_async_copy(
                table_hbm.at[idx_v.at[c]], rows[b], gsem[b]
            )

        def write_copy(b, c):
            return pltpu.make_async_copy(
                rows[b], out_hbm.at[pl.ds(base + c * CHUNK, CHUNK)], wsem[b]
            )

        for b in range(DEPTH):
            gather_copy(b, b).start()

        def body(g, carry):
            for b in range(RING):
                c = g * RING + b
                gather_copy(b, c).wait()
                write_copy(b, c).start()
                bg = (b + DEPTH) % RING
                nxt = c + DEPTH

                @pl.when(nxt < n_chunks)
                def _():
                    @pl.when(nxt >= RING)
                    def _():
                        write_copy(bg, nxt - RING).wait()

                    gather_copy(bg, nxt).start()

            return carry

        lax.fori_loop(0, n_groups, body, 0)
        for b in range(RING):
            write_copy(b, n_chunks - RING + b).wait()

    return gather


def kernel(table, type_index):
    bsz, hist = type_index.shape
    n_rows = bsz * hist
    idx = type_index.astype(jnp.int32).reshape(NW, n_rows // NW // CHUNK, CHUNK)
    out = _make_gather(n_rows)(table, idx)
    return out.reshape(bsz, hist, EMBED)
